# baseline (speedup 1.0000x reference)
import functools

import jax
import jax.numpy as jnp
from jax import lax
from jax.experimental import pallas as pl
from jax.experimental.pallas import tpu as pltpu
from jax.experimental.pallas import tpu_sc as plsc

_VOCAB2 = 1002
_NPOS = 24
_EMBED = 128
_NC, _NS = 2, 16
_NW = _NC * _NS
_BATCH = 1024
_HIST = 200
_N = _BATCH * _HIST
_BR_W = _BATCH // _NW
_CS = 40
_CPR = _HIST // _CS
_NCHK = _BR_W * _CPR
_NBUF = 10
_LOOK = 5
_NGRP = _NCHK // _NBUF


def _build_comb(W_word, W_pos, tokens, pos):
    def body(w_ref, p_ref, t_ref, q_ref, comb_ref, cidx_ref):
        comb_ref[...] = w_ref[...][:, None, :] + p_ref[...][None, :, :]
        cidx_ref[...] = t_ref[...] * _NPOS + q_ref[...]

    comb, cidx = pl.pallas_call(
        body,
        out_shape=[
            jax.ShapeDtypeStruct((_VOCAB2, _NPOS, _EMBED), jnp.float32),
            jax.ShapeDtypeStruct((_BATCH, _HIST), jnp.int32),
        ],
    )(W_word, W_pos, tokens, pos)
    return comb.reshape(_VOCAB2 * _NPOS, _EMBED), cidx


def _sc_lookup(cidx, wcomb):
    mesh = plsc.VectorSubcoreMesh(
        core_axis_name="c", subcore_axis_name="s",
        num_cores=_NC, num_subcores=_NS)

    @functools.partial(
        pl.kernel,
        out_type=jax.ShapeDtypeStruct((_N, _EMBED), jnp.float32),
        mesh=mesh,
        scratch_types=[
            pltpu.VMEM((_BR_W, _HIST), jnp.int32),
            pltpu.VMEM((_BR_W * _HIST,), jnp.int32),
            [pltpu.VMEM((_CS, _EMBED), jnp.float32) for _ in range(_NBUF)],
            [pltpu.SemaphoreType.DMA for _ in range(_NBUF)],
            [pltpu.SemaphoreType.DMA for _ in range(_NBUF)],
        ],
    )
    def k(cidx_hbm, comb_hbm, out_hbm, cidx_v, cidx_f, rows, gsem, ssem):
        c = lax.axis_index("c")
        s = lax.axis_index("s")
        wid = s * _NC + c
        rb = wid * _BR_W
        ob = rb * _HIST

        pltpu.sync_copy(cidx_hbm.at[pl.ds(rb, _BR_W)], cidx_v)

        starts = list(range(0, _HIST - 16, 16)) + [_HIST - 16]

        def repack(r, carry):
            for st in starts:
                cidx_f[pl.ds(r * _HIST + st, 16)] = cidx_v[r, pl.ds(st, 16)]
            return carry
        lax.fori_loop(0, _BR_W, repack, 0)

        def start_gather(b, t):
            pltpu.async_copy(comb_hbm.at[cidx_f.at[pl.ds(t * _CS, _CS)]],
                             rows[b], gsem[b])

        def wait_gather(b):
            pltpu.make_async_copy(comb_hbm.at[cidx_f.at[pl.ds(0, _CS)]],
                                  rows[b], gsem[b]).wait()

        def start_scatter(b, t):
            pltpu.async_copy(rows[b], out_hbm.at[pl.ds(ob + t * _CS, _CS)],
                             ssem[b])

        def wait_scatter(b):
            pltpu.make_async_copy(rows[b], out_hbm.at[pl.ds(0, _CS)],
                                  ssem[b]).wait()

        for b in range(_LOOK):
            start_gather(b, b)

        def group(g, carry):
            base = g * _NBUF
            for b in range(_NBUF):
                t = base + b
                bg = (b + _LOOK) % _NBUF

                @pl.when(t + _LOOK < _NCHK)
                def _():
                    @pl.when(t >= _NBUF - _LOOK)
                    def _():
                        wait_scatter(bg)
                    start_gather(bg, t + _LOOK)

                wait_gather(b)
                start_scatter(b, t)
            return carry
        lax.fori_loop(0, _NGRP, group, 0)

        for b in range(_NBUF):
            wait_scatter(b)

    return k(cidx, wcomb)


def kernel(tokens, pos, W_word, W_pos):
    wcomb, cidx = _build_comb(W_word, W_pos,
                              tokens.astype(jnp.int32), pos.astype(jnp.int32))
    out = _sc_lookup(cidx, wcomb)
    return out.reshape(_BATCH, _HIST, _EMBED)


# speedup vs baseline: 1.0020x; 1.0020x over previous
import functools

import jax
import jax.numpy as jnp
from jax import lax
from jax.experimental import pallas as pl
from jax.experimental.pallas import tpu as pltpu
from jax.experimental.pallas import tpu_sc as plsc

_VOCAB2 = 1002
_NPOS = 24
_EMBED = 128
_NC, _NS = 2, 16
_NW = _NC * _NS
_BATCH = 1024
_HIST = 200
_N = _BATCH * _HIST
_BR_W = _BATCH // _NW
_NBUF = 4
_LOOK = 2
_NGRP = _BR_W // _NBUF
_SPLIT = 128
_REM = _HIST - _SPLIT


def _build_comb(W_word, W_pos, tokens, pos):
    def body(w_ref, p_ref, t_ref, q_ref, comb_ref, cidx_ref):
        comb_ref[...] = w_ref[...][:, None, :] + p_ref[...][None, :, :]
        cidx_ref[...] = t_ref[...] * _NPOS + q_ref[...]

    grid = 8
    vb = 128
    bb = _BATCH // grid
    comb, cidx = pl.pallas_call(
        body,
        grid=(grid,),
        in_specs=[
            pl.BlockSpec((vb, _EMBED), lambda i: (i, 0)),
            pl.BlockSpec((_NPOS, _EMBED), lambda i: (0, 0)),
            pl.BlockSpec((bb, _HIST), lambda i: (i, 0)),
            pl.BlockSpec((bb, _HIST), lambda i: (i, 0)),
        ],
        out_specs=[
            pl.BlockSpec((vb, _NPOS, _EMBED), lambda i: (i, 0, 0)),
            pl.BlockSpec((bb, _HIST), lambda i: (i, 0)),
        ],
        out_shape=[
            jax.ShapeDtypeStruct((_VOCAB2, _NPOS, _EMBED), jnp.float32),
            jax.ShapeDtypeStruct((_BATCH, _HIST), jnp.int32),
        ],
    )(W_word, W_pos, tokens, pos)
    return comb.reshape(_VOCAB2 * _NPOS, _EMBED), cidx


def _sc_lookup(cidx, wcomb):
    mesh = plsc.VectorSubcoreMesh(
        core_axis_name="c", subcore_axis_name="s",
        num_cores=_NC, num_subcores=_NS)

    @functools.partial(
        pl.kernel,
        out_type=jax.ShapeDtypeStruct((_N, _EMBED), jnp.float32),
        mesh=mesh,
        scratch_types=[
            pltpu.VMEM((_BR_W, _HIST), jnp.int32),
            [pltpu.VMEM((_HIST, _EMBED), jnp.float32) for _ in range(_NBUF)],
            [pltpu.SemaphoreType.DMA for _ in range(_NBUF)],
            [pltpu.SemaphoreType.DMA for _ in range(_NBUF)],
        ],
    )
    def k(cidx_hbm, comb_hbm, out_hbm, cidx_v, rows, gsem, ssem):
        c = lax.axis_index("c")
        s = lax.axis_index("s")
        wid = s * _NC + c
        rb = wid * _BR_W

        pltpu.sync_copy(cidx_hbm.at[pl.ds(rb, _BR_W)], cidx_v)

        def start_gather(b, r):
            pltpu.async_copy(comb_hbm.at[cidx_v.at[r, pl.ds(0, _SPLIT)]],
                             rows[b].at[pl.ds(0, _SPLIT)], gsem[b])
            pltpu.async_copy(comb_hbm.at[cidx_v.at[r, pl.ds(_SPLIT, _REM)]],
                             rows[b].at[pl.ds(_SPLIT, _REM)], gsem[b])

        def wait_gather(b):
            pltpu.make_async_copy(comb_hbm.at[cidx_v.at[0, pl.ds(0, _SPLIT)]],
                                  rows[b].at[pl.ds(0, _SPLIT)], gsem[b]).wait()
            pltpu.make_async_copy(comb_hbm.at[cidx_v.at[0, pl.ds(_SPLIT, _REM)]],
                                  rows[b].at[pl.ds(_SPLIT, _REM)], gsem[b]).wait()

        def start_scatter(b, r):
            pltpu.async_copy(rows[b],
                             out_hbm.at[pl.ds((rb + r) * _HIST, _HIST)], ssem[b])

        def wait_scatter(b):
            pltpu.make_async_copy(rows[b], out_hbm.at[pl.ds(0, _HIST)],
                                  ssem[b]).wait()

        for b in range(_LOOK):
            start_gather(b, b)

        def group(g, carry):
            base = g * _NBUF
            for b in range(_NBUF):
                r = base + b
                bg = (b + _LOOK) % _NBUF

                @pl.when(r + _LOOK < _BR_W)
                def _():
                    @pl.when(r >= _NBUF - _LOOK)
                    def _():
                        wait_scatter(bg)
                    start_gather(bg, r + _LOOK)

                wait_gather(b)
                start_scatter(b, r)
            return carry
        lax.fori_loop(0, _NGRP, group, 0)

        for b in range(_NBUF):
            wait_scatter(b)

    return k(cidx, wcomb)


def kernel(tokens, pos, W_word, W_pos):
    wcomb, cidx = _build_comb(W_word, W_pos,
                              tokens.astype(jnp.int32), pos.astype(jnp.int32))
    out = _sc_lookup(cidx, wcomb)
    return out.reshape(_BATCH, _HIST, _EMBED)


# speedup vs baseline: 1.0201x; 1.0180x over previous
import functools

import jax
import jax.numpy as jnp
from jax import lax
from jax.experimental import pallas as pl
from jax.experimental.pallas import tpu as pltpu
from jax.experimental.pallas import tpu_sc as plsc

_VOCAB2 = 1002
_NPOS = 24
_EMBED = 128
_NC, _NS = 2, 16
_NW = _NC * _NS
_BATCH = 1024
_HIST = 200
_N = _BATCH * _HIST
_BR_W = _BATCH // _NW
_NBUF = 4
_LOOK = 2
_NGRP = _BR_W // _NBUF
_SPLIT = 128
_REM = _HIST - _SPLIT


def _build_comb(W_word, W_pos, tokens, pos):
    def body(w_ref, p_ref, t_ref, q_ref, comb_ref, cidx_ref):
        comb_ref[...] = w_ref[...][:, None, :] + p_ref[...][None, :, :]
        cidx_ref[...] = t_ref[...] * _NPOS + q_ref[...]

    comb, cidx = pl.pallas_call(
        body,
        out_shape=[
            jax.ShapeDtypeStruct((_VOCAB2, _NPOS, _EMBED), jnp.float32),
            jax.ShapeDtypeStruct((_BATCH, _HIST), jnp.int32),
        ],
    )(W_word, W_pos, tokens, pos)
    return comb.reshape(_VOCAB2 * _NPOS, _EMBED), cidx


def _sc_lookup(cidx, wcomb):
    mesh = plsc.VectorSubcoreMesh(
        core_axis_name="c", subcore_axis_name="s",
        num_cores=_NC, num_subcores=_NS)

    @functools.partial(
        pl.kernel,
        out_type=jax.ShapeDtypeStruct((_N, _EMBED), jnp.float32),
        mesh=mesh,
        scratch_types=[
            pltpu.VMEM((_BR_W, _HIST), jnp.int32),
            [pltpu.VMEM((_HIST, _EMBED), jnp.float32) for _ in range(_NBUF)],
            [pltpu.SemaphoreType.DMA for _ in range(_NBUF)],
            [pltpu.SemaphoreType.DMA for _ in range(_NBUF)],
            [pltpu.SemaphoreType.DMA for _ in range(_NBUF)],
        ],
    )
    def k(cidx_hbm, comb_hbm, out_hbm, cidx_v, rows, gsemA, gsemB, ssem):
        c = lax.axis_index("c")
        s = lax.axis_index("s")
        wid = s * _NC + c
        rb = wid * _BR_W

        pltpu.sync_copy(cidx_hbm.at[pl.ds(rb, _BR_W)], cidx_v)

        slA, slB = pl.ds(0, _SPLIT), pl.ds(_SPLIT, _REM)

        def start_gather(b, r):
            pltpu.async_copy(comb_hbm.at[cidx_v.at[r, slA]],
                             rows[b].at[slA], gsemA[b])
            pltpu.async_copy(comb_hbm.at[cidx_v.at[r, slB]],
                             rows[b].at[slB], gsemB[b])

        def move(b, r):
            pltpu.make_async_copy(comb_hbm.at[cidx_v.at[0, slA]],
                                  rows[b].at[slA], gsemA[b]).wait()
            pltpu.async_copy(rows[b].at[slA],
                             out_hbm.at[pl.ds((rb + r) * _HIST, _SPLIT)],
                             ssem[b])
            pltpu.make_async_copy(comb_hbm.at[cidx_v.at[0, slB]],
                                  rows[b].at[slB], gsemB[b]).wait()
            pltpu.async_copy(rows[b].at[slB],
                             out_hbm.at[pl.ds((rb + r) * _HIST + _SPLIT, _REM)],
                             ssem[b])

        def wait_scatter(b):
            pltpu.make_async_copy(rows[b].at[slA],
                                  out_hbm.at[pl.ds(0, _SPLIT)], ssem[b]).wait()
            pltpu.make_async_copy(rows[b].at[slB],
                                  out_hbm.at[pl.ds(0, _REM)], ssem[b]).wait()

        for b in range(_LOOK):
            start_gather(b, b)

        def group(g, carry):
            base = g * _NBUF
            for b in range(_NBUF):
                r = base + b
                bg = (b + _LOOK) % _NBUF

                @pl.when(r + _LOOK < _BR_W)
                def _():
                    @pl.when(r >= _NBUF - _LOOK)
                    def _():
                        wait_scatter(bg)
                    start_gather(bg, r + _LOOK)

                move(b, r)
            return carry
        lax.fori_loop(0, _NGRP, group, 0)

        for b in range(_NBUF):
            wait_scatter(b)

    return k(cidx, wcomb)


def kernel(tokens, pos, W_word, W_pos):
    wcomb, cidx = _build_comb(W_word, W_pos,
                              tokens.astype(jnp.int32), pos.astype(jnp.int32))
    out = _sc_lookup(cidx, wcomb)
    return out.reshape(_BATCH, _HIST, _EMBED)


# speedup vs baseline: 1.0208x; 1.0008x over previous
import functools

import jax
import jax.numpy as jnp
from jax import lax
from jax.experimental import pallas as pl
from jax.experimental.pallas import tpu as pltpu
from jax.experimental.pallas import tpu_sc as plsc

_VOCAB2 = 1002
_NPOS = 24
_EMBED = 128
_NC, _NS = 2, 16
_NW = _NC * _NS
_BATCH = 1024
_HIST = 200
_N = _BATCH * _HIST
_BR_W = _BATCH // _NW
_NBUF = 4
_LOOK = 3
_NGRP = _BR_W // _NBUF
_SPLIT = 128
_REM = _HIST - _SPLIT


def _build_comb(W_word, W_pos, tokens, pos):
    def body(w_ref, p_ref, t_ref, q_ref, comb_ref, cidx_ref):
        comb_ref[...] = w_ref[...][:, None, :] + p_ref[...][None, :, :]
        cidx_ref[...] = t_ref[...] * _NPOS + q_ref[...]

    comb, cidx = pl.pallas_call(
        body,
        out_shape=[
            jax.ShapeDtypeStruct((_VOCAB2, _NPOS, _EMBED), jnp.float32),
            jax.ShapeDtypeStruct((_BATCH, _HIST), jnp.int32),
        ],
    )(W_word, W_pos, tokens, pos)
    return comb.reshape(_VOCAB2 * _NPOS, _EMBED), cidx


def _sc_lookup(cidx, wcomb):
    mesh = plsc.VectorSubcoreMesh(
        core_axis_name="c", subcore_axis_name="s",
        num_cores=_NC, num_subcores=_NS)

    @functools.partial(
        pl.kernel,
        out_type=jax.ShapeDtypeStruct((_N, _EMBED), jnp.float32),
        mesh=mesh,
        scratch_types=[
            pltpu.VMEM((_BR_W, _HIST), jnp.int32),
            [pltpu.VMEM((_HIST, _EMBED), jnp.float32) for _ in range(_NBUF)],
            [pltpu.SemaphoreType.DMA for _ in range(_NBUF)],
            [pltpu.SemaphoreType.DMA for _ in range(_NBUF)],
            [pltpu.SemaphoreType.DMA for _ in range(_NBUF)],
        ],
    )
    def k(cidx_hbm, comb_hbm, out_hbm, cidx_v, rows, gsemA, gsemB, ssem):
        c = lax.axis_index("c")
        s = lax.axis_index("s")
        wid = s * _NC + c
        rb = wid * _BR_W

        pltpu.sync_copy(cidx_hbm.at[pl.ds(rb, _BR_W)], cidx_v)

        slA, slB = pl.ds(0, _SPLIT), pl.ds(_SPLIT, _REM)

        def start_gather(b, r):
            pltpu.async_copy(comb_hbm.at[cidx_v.at[r, slA]],
                             rows[b].at[slA], gsemA[b])
            pltpu.async_copy(comb_hbm.at[cidx_v.at[r, slB]],
                             rows[b].at[slB], gsemB[b])

        def move(b, r):
            pltpu.make_async_copy(comb_hbm.at[cidx_v.at[0, slA]],
                                  rows[b].at[slA], gsemA[b]).wait()
            pltpu.async_copy(rows[b].at[slA],
                             out_hbm.at[pl.ds((rb + r) * _HIST, _SPLIT)],
                             ssem[b])
            pltpu.make_async_copy(comb_hbm.at[cidx_v.at[0, slB]],
                                  rows[b].at[slB], gsemB[b]).wait()
            pltpu.async_copy(rows[b].at[slB],
                             out_hbm.at[pl.ds((rb + r) * _HIST + _SPLIT, _REM)],
                             ssem[b])

        def wait_scatter(b):
            pltpu.make_async_copy(rows[b].at[slA],
                                  out_hbm.at[pl.ds(0, _SPLIT)], ssem[b]).wait()
            pltpu.make_async_copy(rows[b].at[slB],
                                  out_hbm.at[pl.ds(0, _REM)], ssem[b]).wait()

        for b in range(_LOOK):
            start_gather(b, b)

        def group(g, carry):
            base = g * _NBUF
            for b in range(_NBUF):
                r = base + b
                bg = (b + _LOOK) % _NBUF

                @pl.when(r + _LOOK < _BR_W)
                def _():
                    @pl.when(r >= _NBUF - _LOOK)
                    def _():
                        wait_scatter(bg)
                    start_gather(bg, r + _LOOK)

                move(b, r)
            return carry
        lax.fori_loop(0, _NGRP, group, 0)

        for b in range(_NBUF):
            wait_scatter(b)

    return k(cidx, wcomb)


def kernel(tokens, pos, W_word, W_pos):
    wcomb, cidx = _build_comb(W_word, W_pos,
                              tokens.astype(jnp.int32), pos.astype(jnp.int32))
    out = _sc_lookup(cidx, wcomb)
    return out.reshape(_BATCH, _HIST, _EMBED)
